# Initial kernel scaffold; baseline (speedup 1.0000x reference)
import jax
import jax.numpy as jnp
from jax.experimental import pallas as pl


def kernel(input, indices):
    raise NotImplementedError("write your pallas kernel here")


# speedup vs baseline: 3.0977x; 3.0977x over previous
import functools

import jax
import jax.numpy as jnp
from jax import lax
from jax.experimental import pallas as pl
from jax.experimental.pallas import tpu as pltpu
from jax.experimental.pallas import tpu_sc as plsc

O, I, ORI, KH, KW, R = 256, 256, 8, 3, 3, 8
E = ORI * KH * KW
ROW = I * E
NC, NS = 2, 16
NW = NC * NS
PPW = O // NW
NCHUNK = ROW // 144


def _arf_body(x_hbm, idxt_hbm, out_hbm, x_v, o_v, pat_v, sem):
    wid = lax.axis_index("s") * NC + lax.axis_index("c")

    for k in range(R):
        pltpu.sync_copy(idxt_hbm.at[pl.ds(k * E, E)], pat_v.at[pl.ds(k * 144, 72)])
        pltpu.sync_copy(idxt_hbm.at[pl.ds(k * E, E)], pat_v.at[pl.ds(k * 144 + 72, 72)])
    ramp = lax.iota(jnp.int32, 16)
    half = jnp.where(ramp >= 8, jnp.full((16,), 72, jnp.int32),
                     jnp.zeros((16,), jnp.int32))
    for k in range(R):
        b = k * 144
        pat_v[pl.ds(b + 64, 16)] = pat_v[pl.ds(b + 64, 16)] + half
        for v in range(5, 9):
            pat_v[pl.ds(b + v * 16, 16)] = pat_v[pl.ds(b + v * 16, 16)] + 72

    for ii in range(PPW):
        i = wid * PPW + ii
        pltpu.sync_copy(x_hbm.at[i], x_v)
        for k in range(R):
            pats = [pat_v[pl.ds(k * 144 + v * 16, 16)] for v in range(9)]

            def body(m, c, pats=pats):
                base = m * 144
                for v in range(9):
                    val = x_v[pl.ds(base + v * 16, 16)]
                    plsc.store_scatter(o_v, [pats[v] + base], val)
                return c

            lax.fori_loop(0, NCHUNK, body, 0)
            pltpu.sync_copy(o_v, out_hbm.at[i * R + k])


@functools.partial(jax.jit, static_argnames=("interpret",))
def _arf(x, idxt, interpret=False):
    mesh = plsc.VectorSubcoreMesh(core_axis_name="c", subcore_axis_name="s",
                                  num_cores=NC, num_subcores=NS)
    f = pl.kernel(
        _arf_body,
        out_type=jax.ShapeDtypeStruct((O * R, ROW), jnp.float32),
        mesh=mesh,
        scratch_types=[
            pltpu.VMEM((ROW,), jnp.float32),
            pltpu.VMEM((ROW,), jnp.float32),
            pltpu.VMEM((R * 144,), jnp.int32),
            pltpu.SemaphoreType.DMA,
        ],
        compiler_params=pltpu.CompilerParams(needs_layout_passes=False),
        interpret=interpret,
    )
    return f(x, idxt)


def kernel(input, indices):
    x = input.reshape(O, ROW)
    idxt = indices.reshape(E, R).T.astype(jnp.int32).reshape(-1)
    out = _arf(x, idxt)
    return out.reshape(O * R, I * ORI, KH, KW)


# speedup vs baseline: 3.3104x; 1.0687x over previous
import functools

import jax
import jax.numpy as jnp
from jax import lax
from jax.experimental import pallas as pl
from jax.experimental.pallas import tpu as pltpu
from jax.experimental.pallas import tpu_sc as plsc

O, I, ORI, KH, KW, R = 256, 256, 8, 3, 3, 8
E = ORI * KH * KW
ROW = I * E
NC, NS = 2, 16
NW = NC * NS
PPW = O // NW
NCHUNK = ROW // 144


def _arf_body(x_hbm, idxt_hbm, out_hbm, x_v, o_v, pat_v,
              sem_x0, sem_x1, sem_o0, sem_o1):
    wid = lax.axis_index("s") * NC + lax.axis_index("c")
    sem_x = (sem_x0, sem_x1)
    sem_o = (sem_o0, sem_o1)

    for k in range(R):
        pltpu.sync_copy(idxt_hbm.at[pl.ds(k * E, E)], pat_v.at[pl.ds(k * 144, 72)])
        pltpu.sync_copy(idxt_hbm.at[pl.ds(k * E, E)], pat_v.at[pl.ds(k * 144 + 72, 72)])
    ramp = lax.iota(jnp.int32, 16)
    half = jnp.where(ramp >= 8, jnp.full((16,), 72, jnp.int32),
                     jnp.zeros((16,), jnp.int32))
    for k in range(R):
        b = k * 144
        pat_v[pl.ds(b + 64, 16)] = pat_v[pl.ds(b + 64, 16)] + half
        for v in range(5, 9):
            pat_v[pl.ds(b + v * 16, 16)] = pat_v[pl.ds(b + v * 16, 16)] + 72

    xdesc = [None, None]
    odesc = [None] * (PPW * R)
    i0 = wid * PPW
    xdesc[0] = pltpu.async_copy(x_hbm.at[i0], x_v.at[pl.ds(0, ROW)], sem_x[0])
    for ii in range(PPW):
        i = i0 + ii
        xdesc[ii % 2].wait()
        if ii + 1 < PPW:
            nb = (ii + 1) % 2
            xdesc[nb] = pltpu.async_copy(x_hbm.at[i + 1],
                                         x_v.at[pl.ds(nb * ROW, ROW)], sem_x[nb])
        for k in range(R):
            s = ii * R + k
            b = s % 2
            if s >= 2:
                odesc[s - 2].wait()
            xoff = (ii % 2) * ROW
            ooff = b * ROW
            pats = [pat_v[pl.ds(k * 144 + v * 16, 16)] for v in range(9)]

            def body(m, c, pats=pats, xoff=xoff, ooff=ooff):
                base = m * 144
                for v in range(9):
                    val = x_v[pl.ds(xoff + base + v * 16, 16)]
                    plsc.store_scatter(o_v, [pats[v] + ooff + base], val)
                return c

            lax.fori_loop(0, NCHUNK, body, 0)
            odesc[s] = pltpu.async_copy(o_v.at[pl.ds(ooff, ROW)],
                                        out_hbm.at[i * R + k], sem_o[b])
    odesc[PPW * R - 2].wait()
    odesc[PPW * R - 1].wait()


@functools.partial(jax.jit, static_argnames=("interpret",))
def _arf(x, idxt, interpret=False):
    mesh = plsc.VectorSubcoreMesh(core_axis_name="c", subcore_axis_name="s",
                                  num_cores=NC, num_subcores=NS)
    f = pl.kernel(
        _arf_body,
        out_type=jax.ShapeDtypeStruct((O * R, ROW), jnp.float32),
        mesh=mesh,
        scratch_types=[
            pltpu.VMEM((2 * ROW,), jnp.float32),
            pltpu.VMEM((2 * ROW,), jnp.float32),
            pltpu.VMEM((R * 144,), jnp.int32),
            pltpu.SemaphoreType.DMA,
            pltpu.SemaphoreType.DMA,
            pltpu.SemaphoreType.DMA,
            pltpu.SemaphoreType.DMA,
        ],
        compiler_params=pltpu.CompilerParams(needs_layout_passes=False),
        interpret=interpret,
    )
    return f(x, idxt)


def kernel(input, indices):
    x = input.reshape(O, ROW)
    idxt = indices.reshape(E, R).T.astype(jnp.int32).reshape(-1)
    out = _arf(x, idxt)
    return out.reshape(O * R, I * ORI, KH, KW)


# speedup vs baseline: 4.4017x; 1.3297x over previous
import functools

import jax
import jax.numpy as jnp
from jax import lax
from jax.experimental import pallas as pl
from jax.experimental.pallas import tpu as pltpu
from jax.experimental.pallas import tpu_sc as plsc

O, I, ORI, KH, KW, R = 256, 256, 8, 3, 3, 8
E = ORI * KH * KW
ROW = I * E
NC, NS = 2, 16
NW = NC * NS
PPW = O // NW
NCHUNK = ROW // 144


def _arf_body(x_hbm, idxt_hbm, out_hbm, x_v, o_v, pat_v,
              sem_x0, sem_x1, sem_o0, sem_o1):
    wid = lax.axis_index("s") * NC + lax.axis_index("c")
    sem_x = (sem_x0, sem_x1)
    sem_o = (sem_o0, sem_o1)

    for k in range(R):
        pltpu.sync_copy(idxt_hbm.at[pl.ds(k * E, E)], pat_v.at[pl.ds(k * 144, 72)])
        pltpu.sync_copy(idxt_hbm.at[pl.ds(k * E, E)], pat_v.at[pl.ds(k * 144 + 72, 72)])
    ramp = lax.iota(jnp.int32, 16)
    half = jnp.where(ramp >= 8, jnp.full((16,), 72, jnp.int32),
                     jnp.zeros((16,), jnp.int32))
    for k in range(R):
        b = k * 144
        pat_v[pl.ds(b + 64, 16)] = pat_v[pl.ds(b + 64, 16)] + half
        for v in range(5, 9):
            pat_v[pl.ds(b + v * 16, 16)] = pat_v[pl.ds(b + v * 16, 16)] + 72

    xdesc = [None, None]
    odesc = [None] * (PPW * R)
    i0 = wid * PPW
    xdesc[0] = pltpu.async_copy(x_hbm.at[i0], x_v.at[pl.ds(0, ROW)], sem_x[0])
    for ii in range(PPW):
        i = i0 + ii
        xdesc[ii % 2].wait()
        if ii + 1 < PPW:
            nb = (ii + 1) % 2
            xdesc[nb] = pltpu.async_copy(x_hbm.at[i + 1],
                                         x_v.at[pl.ds(nb * ROW, ROW)], sem_x[nb])
        for k in range(R):
            s = ii * R + k
            b = s % 2
            if s >= 2:
                odesc[s - 2].wait()
            xoff = (ii % 2) * ROW
            ooff = b * ROW
            pats = [pat_v[pl.ds(k * 144 + v * 16, 16)] for v in range(9)]

            def body(m, pats=pats, xoff=xoff, ooff=ooff):
                base = m * 144
                vals = [x_v[pl.ds(xoff + base + v * 16, 16)] for v in range(9)]
                for v in range(9):
                    plsc.store_scatter(o_v, [pats[v] + (ooff + base)], vals[v])

            plsc.parallel_loop(0, NCHUNK, unroll=2)(body)
            odesc[s] = pltpu.async_copy(o_v.at[pl.ds(ooff, ROW)],
                                        out_hbm.at[i * R + k], sem_o[b])
    odesc[PPW * R - 2].wait()
    odesc[PPW * R - 1].wait()


@functools.partial(jax.jit, static_argnames=("interpret",))
def _arf(x, idxt, interpret=False):
    mesh = plsc.VectorSubcoreMesh(core_axis_name="c", subcore_axis_name="s",
                                  num_cores=NC, num_subcores=NS)
    f = pl.kernel(
        _arf_body,
        out_type=jax.ShapeDtypeStruct((O * R, ROW), jnp.float32),
        mesh=mesh,
        scratch_types=[
            pltpu.VMEM((2 * ROW,), jnp.float32),
            pltpu.VMEM((2 * ROW,), jnp.float32),
            pltpu.VMEM((R * 144,), jnp.int32),
            pltpu.SemaphoreType.DMA,
            pltpu.SemaphoreType.DMA,
            pltpu.SemaphoreType.DMA,
            pltpu.SemaphoreType.DMA,
        ],
        compiler_params=pltpu.CompilerParams(needs_layout_passes=False),
        interpret=interpret,
    )
    return f(x, idxt)


def kernel(input, indices):
    x = input.reshape(O, ROW)
    idxt = indices.reshape(E, R).T.astype(jnp.int32).reshape(-1)
    out = _arf(x, idxt)
    return out.reshape(O * R, I * ORI, KH, KW)


# speedup vs baseline: 9.0256x; 2.0505x over previous
import functools

import jax
import jax.numpy as jnp
from jax import lax
from jax.experimental import pallas as pl
from jax.experimental.pallas import tpu as pltpu
from jax.experimental.pallas import tpu_sc as plsc

O, I, ORI, KH, KW, R = 256, 256, 8, 3, 3, 8
E = ORI * KH * KW
HW = KH * KW
ROW = I * E
BLK = 16 * R * 128
NC, NS = 2, 16
NW = NC * NS
PPW = O // NW


def _arf_body(x_hbm, lut_hbm, out_hbm, x_v, o_v, pat_v,
              sem_x0, sem_x1, sem_o0, sem_o1):
    wid = lax.axis_index("s") * NC + lax.axis_index("c")
    sem_x = (sem_x0, sem_x1)
    sem_o = (sem_o0, sem_o1)
    pltpu.sync_copy(lut_hbm, pat_v)

    i0 = wid * PPW
    xdesc = [None, None]
    odesc = [None] * (PPW * HW)
    xdesc[0] = pltpu.async_copy(x_hbm.at[i0], x_v.at[pl.ds(0, ROW)], sem_x[0])
    for ii in range(PPW):
        i = i0 + ii
        xdesc[ii % 2].wait()
        if ii + 1 < PPW:
            nb = (ii + 1) % 2
            xdesc[nb] = pltpu.async_copy(x_hbm.at[i + 1],
                                         x_v.at[pl.ds(nb * ROW, ROW)], sem_x[nb])
        xoff = (ii % 2) * ROW
        for hw in range(HW):
            s = ii * HW + hw
            b = s % 2
            if s >= 2:
                odesc[s - 2].wait()
            ooff = b * BLK

            def kbody(k, c, hw=hw, xoff=xoff, ooff=ooff):
                bvec = pat_v[pl.ds(hw * 128 + k * 16, 16)]
                kdst = k * 128 + ooff
                for jt in range(2):
                    def jbody(jgm, bv=bvec, kdst=kdst, jt=jt, xoff=xoff):
                        soff = jt * 1024 + jgm * 16 + xoff
                        doff = jt * 8192 + jgm * 1024 + kdst
                        vals = [plsc.load_gather(x_v, [bv + (soff + jmp * 2)])
                                for jmp in range(8)]
                        for jmp in range(8):
                            o_v[pl.ds(doff + jmp * 16, 16)] = vals[jmp]

                    plsc.parallel_loop(0, 8, unroll=2)(jbody)
                return c

            lax.fori_loop(0, R, kbody, 0)
            odesc[s] = pltpu.async_copy(o_v.at[pl.ds(ooff, BLK)],
                                        out_hbm.at[hw, i], sem_o[b])
    odesc[PPW * HW - 2].wait()
    odesc[PPW * HW - 1].wait()


@functools.partial(jax.jit, static_argnames=("interpret",))
def _arf(x2, lut, interpret=False):
    mesh = plsc.VectorSubcoreMesh(core_axis_name="c", subcore_axis_name="s",
                                  num_cores=NC, num_subcores=NS)
    f = pl.kernel(
        _arf_body,
        out_type=jax.ShapeDtypeStruct((HW, O, BLK), jnp.float32),
        mesh=mesh,
        scratch_types=[
            pltpu.VMEM((2 * ROW,), jnp.float32),
            pltpu.VMEM((2 * BLK,), jnp.float32),
            pltpu.VMEM((HW * R * 16,), jnp.int32),
            pltpu.SemaphoreType.DMA,
            pltpu.SemaphoreType.DMA,
            pltpu.SemaphoreType.DMA,
            pltpu.SemaphoreType.DMA,
        ],
        compiler_params=pltpu.CompilerParams(needs_layout_passes=False),
        interpret=interpret,
    )
    return f(x2, lut)


def kernel(input, indices):
    x2 = (input.reshape(O, 2, 128, ORI, KH, KW)
          .transpose(0, 4, 5, 1, 3, 2).reshape(O, ROW))
    idx2 = indices.reshape(E, R).astype(jnp.int32)
    inv = jnp.argsort(idx2, axis=0)
    base = ((inv % HW) * 2048 + (inv // HW) * 128)
    bt = base.reshape(ORI, HW, R).transpose(1, 2, 0)
    lut = (bt[:, :, None, :] +
           jnp.arange(2, dtype=jnp.int32)[None, None, :, None]).reshape(-1)
    out6 = _arf(x2, lut)
    return (out6.reshape(KH, KW, O, 16, R, 128)
            .transpose(2, 4, 3, 5, 0, 1).reshape(O * R, I * ORI, KH, KW))


# speedup vs baseline: 12.3132x; 1.3643x over previous
import functools

import jax
import jax.numpy as jnp
from jax import lax
from jax.experimental import pallas as pl
from jax.experimental.pallas import tpu as pltpu
from jax.experimental.pallas import tpu_sc as plsc

O, I, ORI, KH, KW, R = 256, 256, 8, 3, 3, 8
E = ORI * KH * KW
HW = KH * KW
ROW = I * E
BLK = 16 * R * 128
NC, NS = 2, 16
NW = NC * NS
PPW = O // NW


def _arf_body(x_hbm, slut_hbm, dlut_hbm, out_hbm, x_v, o_v, spat_v, dpat_v,
              sem_x0, sem_x1, sem_o0, sem_o1):
    wid = lax.axis_index("s") * NC + lax.axis_index("c")
    sem_x = (sem_x0, sem_x1)
    sem_o = (sem_o0, sem_o1)
    pltpu.sync_copy(slut_hbm, spat_v)
    pltpu.sync_copy(dlut_hbm, dpat_v)

    i0 = wid * PPW
    xdesc = [None, None]
    odesc = [None] * (PPW * HW)
    xdesc[0] = pltpu.async_copy(x_hbm.at[i0], x_v.at[pl.ds(0, ROW)], sem_x[0])
    for ii in range(PPW):
        i = i0 + ii
        xdesc[ii % 2].wait()
        if ii + 1 < PPW:
            nb = (ii + 1) % 2
            xdesc[nb] = pltpu.async_copy(x_hbm.at[i + 1],
                                         x_v.at[pl.ds(nb * ROW, ROW)], sem_x[nb])
        xoff = (ii % 2) * ROW
        for hw in range(HW):
            s = ii * HW + hw
            b = s % 2
            if s >= 2:
                odesc[s - 2].wait()
            ooff = b * BLK

            def kbody(k, c, hw=hw, xoff=xoff, ooff=ooff):
                svecs = [spat_v[pl.ds((hw * R + k) * 128 + t * 16, 16)]
                         for t in range(8)]
                dvecs = [dpat_v[pl.ds(k * 128 + t * 16, 16)] for t in range(8)]
                def jbody(jg, svecs=svecs, dvecs=dvecs, xoff=xoff, ooff=ooff):
                    soff = ((jg >> 3) << 10) + ((jg & 7) << 4) + xoff
                    doff = (jg << 10) + ooff
                    vals = [plsc.load_gather(x_v, [svecs[t] + soff])
                            for t in range(8)]
                    for t in range(8):
                        plsc.store_scatter(o_v, [dvecs[t] + doff], vals[t])

                plsc.parallel_loop(0, 16, unroll=2)(jbody)
                return c

            lax.fori_loop(0, R, kbody, 0)
            odesc[s] = pltpu.async_copy(o_v.at[pl.ds(ooff, BLK)],
                                        out_hbm.at[hw, i], sem_o[b])
    odesc[PPW * HW - 2].wait()
    odesc[PPW * HW - 1].wait()


@functools.partial(jax.jit, static_argnames=("interpret",))
def _arf(x2, slut, dlut, interpret=False):
    mesh = plsc.VectorSubcoreMesh(core_axis_name="c", subcore_axis_name="s",
                                  num_cores=NC, num_subcores=NS)
    f = pl.kernel(
        _arf_body,
        out_type=jax.ShapeDtypeStruct((HW, O, BLK), jnp.float32),
        mesh=mesh,
        scratch_types=[
            pltpu.VMEM((2 * ROW,), jnp.float32),
            pltpu.VMEM((2 * BLK,), jnp.float32),
            pltpu.VMEM((HW * R * 128,), jnp.int32),
            pltpu.VMEM((R * 128,), jnp.int32),
            pltpu.SemaphoreType.DMA,
            pltpu.SemaphoreType.DMA,
            pltpu.SemaphoreType.DMA,
            pltpu.SemaphoreType.DMA,
        ],
        compiler_params=pltpu.CompilerParams(needs_layout_passes=False),
        interpret=interpret,
    )
    return f(x2, slut, dlut)


def kernel(input, indices):
    x2 = (input.reshape(O, 2, 128, ORI, KH, KW)
          .transpose(0, 4, 5, 1, 3, 2).reshape(O, ROW))
    idx2 = indices.reshape(E, R).astype(jnp.int32)
    inv = jnp.argsort(idx2, axis=0)
    base = ((inv % HW) * 2048 + (inv // HW) * 128)
    bt = base.reshape(ORI, HW, R).transpose(1, 2, 0)
    q = jnp.arange(16, dtype=jnp.int32)
    t = jnp.arange(8, dtype=jnp.int32)
    sig = (2 * (q[None, :] % 8) + q[None, :] // 8 + 2 * t[:, None]) % 16
    o2q = q % 8
    slut = (bt[:, :, None, o2q] + sig[None, None, :, :]).reshape(-1)
    dlut = (jnp.arange(8, dtype=jnp.int32)[:, None, None] * 128 +
            sig[None] * 8 + o2q[None, None]).reshape(-1)
    out6 = _arf(x2, slut, dlut)
    return (out6.reshape(KH, KW, O, 16, R, 128)
            .transpose(2, 4, 3, 5, 0, 1).reshape(O * R, I * ORI, KH, KW))


# speedup vs baseline: 28.9447x; 2.3507x over previous
import functools

import jax
import jax.numpy as jnp
from jax import lax
from jax.experimental import pallas as pl
from jax.experimental.pallas import tpu as pltpu
from jax.experimental.pallas import tpu_sc as plsc

O, I, ORI, KH, KW, R = 256, 256, 8, 3, 3, 8
E = ORI * KH * KW
HW = KH * KW
ROW = I * E
BLK = 16 * R * 128
NC, NS = 2, 16
NW = NC * NS
PPW = O // NW


def _arf_body(x_hbm, slut_hbm, dlut_hbm, out_hbm, x_v, o_v, spat_v, dpat_v,
              sem_x0, sem_x1, sem_o0, sem_o1):
    wid = lax.axis_index("s") * NC + lax.axis_index("c")
    sem_x = (sem_x0, sem_x1)
    sem_o = (sem_o0, sem_o1)
    pltpu.sync_copy(slut_hbm, spat_v)
    pltpu.sync_copy(dlut_hbm, dpat_v)

    i0 = wid * PPW
    xdesc = [None, None]
    odesc = [None] * (PPW * HW)
    xdesc[0] = pltpu.async_copy(x_hbm.at[pl.ds(i0 * ROW, ROW)],
                                x_v.at[pl.ds(0, ROW)], sem_x[0])
    for ii in range(PPW):
        i = i0 + ii
        xdesc[ii % 2].wait()
        if ii + 1 < PPW:
            nb = (ii + 1) % 2
            xdesc[nb] = pltpu.async_copy(x_hbm.at[pl.ds((i + 1) * ROW, ROW)],
                                         x_v.at[pl.ds(nb * ROW, ROW)], sem_x[nb])
        xoff = (ii % 2) * ROW
        for hw in range(HW):
            s = ii * HW + hw
            b = s % 2
            if s >= 2:
                odesc[s - 2].wait()
            ooff = b * BLK

            def kbody(k, c, hw=hw, xoff=xoff, ooff=ooff):
                svecs = [spat_v[pl.ds((hw * R + k) * 128 + t * 16, 16)]
                         for t in range(8)]
                dvecs = [dpat_v[pl.ds(k * 128 + t * 16, 16)] for t in range(8)]
                def jbody(jg, svecs=svecs, dvecs=dvecs, xoff=xoff, ooff=ooff):
                    soff = ((jg >> 3) << 10) + ((jg & 7) << 4) + xoff
                    doff = (jg << 10) + ooff
                    vals = [plsc.load_gather(x_v, [svecs[t] + soff])
                            for t in range(8)]
                    for t in range(8):
                        plsc.store_scatter(o_v, [dvecs[t] + doff], vals[t])

                plsc.parallel_loop(0, 16, unroll=2)(jbody)
                return c

            lax.fori_loop(0, R, kbody, 0)
            odesc[s] = pltpu.async_copy(o_v.at[pl.ds(ooff, BLK)],
                                        out_hbm.at[pl.ds((hw * O + i) * BLK, BLK)],
                                        sem_o[b])
    odesc[PPW * HW - 2].wait()
    odesc[PPW * HW - 1].wait()


@functools.partial(jax.jit, static_argnames=("interpret",))
def _arf(x2, slut, dlut, interpret=False):
    mesh = plsc.VectorSubcoreMesh(core_axis_name="c", subcore_axis_name="s",
                                  num_cores=NC, num_subcores=NS)
    f = pl.kernel(
        _arf_body,
        out_type=jax.ShapeDtypeStruct((HW * O * BLK,), jnp.float32),
        mesh=mesh,
        scratch_types=[
            pltpu.VMEM((2 * ROW,), jnp.float32),
            pltpu.VMEM((2 * BLK,), jnp.float32),
            pltpu.VMEM((HW * R * 128,), jnp.int32),
            pltpu.VMEM((R * 128,), jnp.int32),
            pltpu.SemaphoreType.DMA,
            pltpu.SemaphoreType.DMA,
            pltpu.SemaphoreType.DMA,
            pltpu.SemaphoreType.DMA,
        ],
        compiler_params=pltpu.CompilerParams(needs_layout_passes=False),
        interpret=interpret,
    )
    return f(x2, slut, dlut)


def kernel(input, indices):
    x2 = (input.reshape(O, 2, 128, ORI, KH, KW)
          .transpose(0, 4, 5, 1, 3, 2).reshape(-1))
    idx2 = indices.reshape(E, R).astype(jnp.int32)
    inv = jnp.argsort(idx2, axis=0)
    base = ((inv % HW) * 2048 + (inv // HW) * 128)
    bt = base.reshape(ORI, HW, R).transpose(1, 2, 0)
    q = jnp.arange(16, dtype=jnp.int32)
    t = jnp.arange(8, dtype=jnp.int32)
    sig = (2 * (q[None, :] % 8) + q[None, :] // 8 + 2 * t[:, None]) % 16
    o2q = q % 8
    slut = (bt[:, :, None, o2q] + sig[None, None, :, :]).reshape(-1)
    dlut = (jnp.arange(8, dtype=jnp.int32)[:, None, None] * 128 +
            sig[None] * 8 + o2q[None, None]).reshape(-1)
    out6 = _arf(x2, slut, dlut)
    return (out6.reshape(KH, KW, O, 16, R, 128)
            .transpose(2, 4, 3, 5, 0, 1).reshape(O * R, I * ORI, KH, KW))


# speedup vs baseline: 29.1161x; 1.0059x over previous
import functools

import jax
import jax.numpy as jnp
from jax import lax
from jax.experimental import pallas as pl
from jax.experimental.pallas import tpu as pltpu
from jax.experimental.pallas import tpu_sc as plsc

O, I, ORI, KH, KW, R = 256, 256, 8, 3, 3, 8
E = ORI * KH * KW
HW = KH * KW
ROW = I * E
BLK = 16 * R * 128
NC, NS = 2, 16
NW = NC * NS
PPW = O // NW


def _arf_body(x_hbm, slut_hbm, dlut_hbm, out_hbm, x_v, o_v, spat_v, dpat_v,
              sem_x0, sem_x1, sem_o0, sem_o1, sem_o2):
    wid = lax.axis_index("s") * NC + lax.axis_index("c")
    sem_x = (sem_x0, sem_x1)
    sem_o = (sem_o0, sem_o1, sem_o2)
    pltpu.sync_copy(slut_hbm, spat_v)
    pltpu.sync_copy(dlut_hbm, dpat_v)

    i0 = wid * PPW
    xdesc = [None, None]
    odesc = [None] * (PPW * HW)
    xdesc[0] = pltpu.async_copy(x_hbm.at[pl.ds(i0 * ROW, ROW)],
                                x_v.at[pl.ds(0, ROW)], sem_x[0])
    for ii in range(PPW):
        i = i0 + ii
        xdesc[ii % 2].wait()
        if ii + 1 < PPW:
            nb = (ii + 1) % 2
            xdesc[nb] = pltpu.async_copy(x_hbm.at[pl.ds((i + 1) * ROW, ROW)],
                                         x_v.at[pl.ds(nb * ROW, ROW)], sem_x[nb])
        xoff = (ii % 2) * ROW
        for hw in range(HW):
            s = ii * HW + hw
            b = s % 3
            if s >= 3:
                odesc[s - 3].wait()
            ooff = b * BLK

            def kbody(k, c, hw=hw, xoff=xoff, ooff=ooff):
                svecs = [spat_v[pl.ds((hw * R + k) * 128 + t * 16, 16)]
                         for t in range(8)]
                dvecs = [dpat_v[pl.ds(k * 128 + t * 16, 16)] for t in range(8)]
                def jbody(jg, svecs=svecs, dvecs=dvecs, xoff=xoff, ooff=ooff):
                    soff = ((jg >> 3) << 10) + ((jg & 7) << 4) + xoff
                    doff = (jg << 10) + ooff
                    vals = [plsc.load_gather(x_v, [svecs[t] + soff])
                            for t in range(8)]
                    for t in range(8):
                        plsc.store_scatter(o_v, [dvecs[t] + doff], vals[t])

                plsc.parallel_loop(0, 16, unroll=2)(jbody)
                return c

            lax.fori_loop(0, R, kbody, 0)
            odesc[s] = pltpu.async_copy(o_v.at[pl.ds(ooff, BLK)],
                                        out_hbm.at[pl.ds((hw * O + i) * BLK, BLK)],
                                        sem_o[b])
    odesc[PPW * HW - 3].wait()
    odesc[PPW * HW - 2].wait()
    odesc[PPW * HW - 1].wait()


@functools.partial(jax.jit, static_argnames=("interpret",))
def _arf(x2, slut, dlut, interpret=False):
    mesh = plsc.VectorSubcoreMesh(core_axis_name="c", subcore_axis_name="s",
                                  num_cores=NC, num_subcores=NS)
    f = pl.kernel(
        _arf_body,
        out_type=jax.ShapeDtypeStruct((HW * O * BLK,), jnp.float32),
        mesh=mesh,
        scratch_types=[
            pltpu.VMEM((2 * ROW,), jnp.float32),
            pltpu.VMEM((3 * BLK,), jnp.float32),
            pltpu.VMEM((HW * R * 128,), jnp.int32),
            pltpu.VMEM((R * 128,), jnp.int32),
            pltpu.SemaphoreType.DMA,
            pltpu.SemaphoreType.DMA,
            pltpu.SemaphoreType.DMA,
            pltpu.SemaphoreType.DMA,
            pltpu.SemaphoreType.DMA,
        ],
        compiler_params=pltpu.CompilerParams(needs_layout_passes=False),
        interpret=interpret,
    )
    return f(x2, slut, dlut)


def kernel(input, indices):
    x2 = (input.reshape(O, 2, 128, ORI, KH, KW)
          .transpose(0, 4, 5, 1, 3, 2).reshape(-1))
    idx2 = indices.reshape(E, R).astype(jnp.int32)
    inv = jnp.argsort(idx2, axis=0)
    base = ((inv % HW) * 2048 + (inv // HW) * 128)
    bt = base.reshape(ORI, HW, R).transpose(1, 2, 0)
    q = jnp.arange(16, dtype=jnp.int32)
    t = jnp.arange(8, dtype=jnp.int32)
    sig = (2 * (q[None, :] % 8) + q[None, :] // 8 + 2 * t[:, None]) % 16
    o2q = q % 8
    slut = (bt[:, :, None, o2q] + sig[None, None, :, :]).reshape(-1)
    dlut = (jnp.arange(8, dtype=jnp.int32)[:, None, None] * 128 +
            sig[None] * 8 + o2q[None, None]).reshape(-1)
    out6 = _arf(x2, slut, dlut)
    return (out6.reshape(KH, KW, O, 16, R, 128)
            .transpose(2, 4, 3, 5, 0, 1).reshape(O * R, I * ORI, KH, KW))
